# baseline (speedup 1.0000x reference)
import functools

import jax
import jax.numpy as jnp
from jax import lax
from jax.experimental import pallas as pl
from jax.experimental.pallas import tpu as pltpu

N = 10000
E = 320000
H = 128


def _mm_kernel(x_ref, w_ref, b_ref, o_ref, *, activation):
    acc = jnp.dot(x_ref[...], w_ref[...], preferred_element_type=jnp.float32)
    acc = acc + b_ref[...]
    if activation == "relu":
        acc = jnp.maximum(acc, 0.0)
    o_ref[...] = acc


def _matmul(x, w_t, b, activation=None, block_m=512):
    m, k = x.shape
    n = w_t.shape[1]
    grid = (pl.cdiv(m, block_m),)
    return pl.pallas_call(
        functools.partial(_mm_kernel, activation=activation),
        grid=grid,
        in_specs=[
            pl.BlockSpec((block_m, k), lambda i: (i, 0)),
            pl.BlockSpec((k, n), lambda i: (0, 0)),
            pl.BlockSpec((1, n), lambda i: (0, 0)),
        ],
        out_specs=pl.BlockSpec((block_m, n), lambda i: (i, 0)),
        out_shape=jax.ShapeDtypeStruct((m, n), jnp.float32),
    )(x, w_t, b.reshape(1, n))


def _mm2_kernel(x_ref, wa_ref, ba_ref, wb_ref, bb_ref, oa_ref, ob_ref):
    x = x_ref[...]
    a = jnp.dot(x, wa_ref[...], preferred_element_type=jnp.float32) + ba_ref[...]
    b = jnp.dot(x, wb_ref[...], preferred_element_type=jnp.float32) + bb_ref[...]
    oa_ref[...] = a
    ob_ref[...] = b


def _matmul2(x, wa_t, ba, wb_t, bb, block_m=512):
    m, k = x.shape
    n = wa_t.shape[1]
    grid = (pl.cdiv(m, block_m),)
    return pl.pallas_call(
        _mm2_kernel,
        grid=grid,
        in_specs=[
            pl.BlockSpec((block_m, k), lambda i: (i, 0)),
            pl.BlockSpec((k, n), lambda i: (0, 0)),
            pl.BlockSpec((1, n), lambda i: (0, 0)),
            pl.BlockSpec((k, n), lambda i: (0, 0)),
            pl.BlockSpec((1, n), lambda i: (0, 0)),
        ],
        out_specs=[
            pl.BlockSpec((block_m, n), lambda i: (i, 0)),
            pl.BlockSpec((block_m, n), lambda i: (i, 0)),
        ],
        out_shape=[
            jax.ShapeDtypeStruct((m, n), jnp.float32),
            jax.ShapeDtypeStruct((m, n), jnp.float32),
        ],
    )(x, wa_t, ba.reshape(1, n), wb_t, bb.reshape(1, n))


def _edge_conv(z, src, dst, w1, b1, w2, b2):
    w1d = w1[:, :H].T
    w1s = w1[:, H:].T
    a_dst, b_src = _matmul2(z, w1d, b1, w1s, jnp.zeros_like(b1))
    u = jnp.maximum(jnp.take(a_dst, dst, axis=0) + jnp.take(b_src, src, axis=0), 0.0)
    m = _matmul(u, w2.T, b2)
    agg = jax.ops.segment_max(m, dst, num_segments=N)
    return jnp.where(jnp.isneginf(agg), 0.0, agg)


def kernel(x, h, edge_index, enc_w, enc_b, conv0_w1, conv0_b1, conv0_w2, conv0_b2, conv1_w1, conv1_b1, conv1_w2, conv1_b2, dec_w, dec_b, dec_w1, dec_b1, head_w, head_b, term_w, term_b):
    src = edge_index[0]
    dst = edge_index[1]
    z = _matmul(jnp.concatenate([x, h], axis=1), enc_w.T, enc_b, activation="relu")
    hh = jnp.maximum(_edge_conv(z, src, dst, conv0_w1, conv0_b1, conv0_w2, conv0_b2), 0.0)
    hh = _edge_conv(hh, src, dst, conv1_w1, conv1_b1, conv1_w2, conv1_b2)
    o = _matmul(jnp.concatenate([hh, z], axis=1), dec_w.T, dec_b, activation="relu")
    o = _matmul(o, dec_w1.T, dec_b1, activation="relu")
    y = jax.nn.sigmoid(o @ head_w.T + head_b)
    h_bar = jnp.mean(hh, axis=0)
    t = jax.nn.sigmoid(h_bar @ term_w.T + term_b)
    return (y, t, hh)
